# Initial kernel scaffold; baseline (speedup 1.0000x reference)
import jax
import jax.numpy as jnp
from jax.experimental import pallas as pl


def kernel(x, Wg, W1, W3, W2, Ws1, Ws3, Ws2):
    raise NotImplementedError("write your pallas kernel here")


# speedup vs baseline: 2.0337x; 2.0337x over previous
import functools

import jax
import jax.numpy as jnp
from jax.experimental import pallas as pl
from jax.experimental.pallas import tpu as pltpu

DIM = 768
NUM_EXPERTS = 8
TOP_K = 2
HID = 2058
S = 2048
G = S * TOP_K
TM = 256
T_TILES = G // TM
NUM_W = T_TILES + NUM_EXPERTS - 1


def _router_body(x_ref, wg_ref, wpair_ref, epair_ref, aux_ref):
    x = x_ref[...]
    logits = jnp.dot(x, wg_ref[...], preferred_element_type=jnp.float32)
    m = jnp.max(logits, axis=1, keepdims=True)
    e = jnp.exp(logits - m)
    p = e / jnp.sum(e, axis=1, keepdims=True)

    idx8 = jax.lax.broadcasted_iota(jnp.int32, p.shape, 1)
    m0 = jnp.max(p, axis=1, keepdims=True)
    i0 = jnp.min(jnp.where(p == m0, idx8, NUM_EXPERTS), axis=1, keepdims=True)
    pm = jnp.where(idx8 == i0, -1.0, p)
    m1 = jnp.max(pm, axis=1, keepdims=True)
    i1 = jnp.min(jnp.where(pm == m1, idx8, NUM_EXPERTS), axis=1, keepdims=True)

    s = m0 + m1
    wpair_ref[:, 0:1] = m0 / s
    wpair_ref[:, 1:2] = m1 / s
    epair_ref[:, 0:1] = i0
    epair_ref[:, 1:2] = i1

    importance = jnp.mean(p, axis=0, keepdims=True)
    load = jnp.mean((idx8 == i0).astype(jnp.float32), axis=0, keepdims=True)
    aux_ref[...] = NUM_EXPERTS * jnp.sum(
        importance * load, axis=1, keepdims=True)


def _router(xf, Wg):
    return pl.pallas_call(
        _router_body,
        out_shape=(
            jax.ShapeDtypeStruct((S, TOP_K), jnp.float32),
            jax.ShapeDtypeStruct((S, TOP_K), jnp.int32),
            jax.ShapeDtypeStruct((1, 1), jnp.float32),
        ),
    )(xf, Wg)


def _gmm_body(meta_ref, xs_ref, wr_ref, w1_ref, w3_ref, w2_ref, ys_ref):
    w = pl.program_id(0)
    rs = meta_ref[2, w]
    re_ = meta_ref[3, w]
    fst = meta_ref[4, w]

    @pl.when(fst == 1)
    def _():
        ys_ref[...] = jnp.zeros_like(ys_ref)

    @pl.when(re_ > rs)
    def _():
        x = xs_ref[...]
        a = jnp.dot(x, w1_ref[0], preferred_element_type=jnp.float32)
        b = jnp.dot(x, w3_ref[0], preferred_element_type=jnp.float32)
        h = a * (1.0 / (1.0 + jnp.exp(-a))) * b
        y = jnp.dot(h, w2_ref[0], preferred_element_type=jnp.float32)
        y = y * wr_ref[...]
        rows = jax.lax.broadcasted_iota(jnp.int32, (TM, 1), 0)
        mask = (rows >= rs) & (rows < re_)
        ys_ref[...] += jnp.where(mask, y, 0.0)


def _gmm(meta, xs, wrow, W1, W3, W2):
    grid_spec = pltpu.PrefetchScalarGridSpec(
        num_scalar_prefetch=1,
        grid=(NUM_W,),
        in_specs=[
            pl.BlockSpec((TM, DIM), lambda w, m: (m[0, w], 0)),
            pl.BlockSpec((TM, 1), lambda w, m: (m[0, w], 0)),
            pl.BlockSpec((1, DIM, HID), lambda w, m: (m[1, w], 0, 0)),
            pl.BlockSpec((1, DIM, HID), lambda w, m: (m[1, w], 0, 0)),
            pl.BlockSpec((1, HID, DIM), lambda w, m: (m[1, w], 0, 0)),
        ],
        out_specs=pl.BlockSpec((TM, DIM), lambda w, m: (m[0, w], 0)),
    )
    return pl.pallas_call(
        _gmm_body,
        grid_spec=grid_spec,
        out_shape=jax.ShapeDtypeStruct((G, DIM), jnp.float32),
        compiler_params=pltpu.CompilerParams(
            dimension_semantics=("arbitrary",)),
    )(meta, xs, wrow.reshape(G, 1), W1, W3, W2)


def _shared_body(x_ref, w1_ref, w3_ref, w2_ref, o_ref):
    x = x_ref[...]
    a = jnp.dot(x, w1_ref[...], preferred_element_type=jnp.float32)
    b = jnp.dot(x, w3_ref[...], preferred_element_type=jnp.float32)
    h = a * (1.0 / (1.0 + jnp.exp(-a))) * b
    o_ref[...] = jnp.dot(h, w2_ref[...], preferred_element_type=jnp.float32)


def _shared(xf, Ws1, Ws3, Ws2):
    nt = S // TM
    return pl.pallas_call(
        _shared_body,
        grid=(nt,),
        in_specs=[
            pl.BlockSpec((TM, DIM), lambda t: (t, 0)),
            pl.BlockSpec((DIM, HID), lambda t: (0, 0)),
            pl.BlockSpec((DIM, HID), lambda t: (0, 0)),
            pl.BlockSpec((HID, DIM), lambda t: (0, 0)),
        ],
        out_specs=pl.BlockSpec((TM, DIM), lambda t: (t, 0)),
        out_shape=jax.ShapeDtypeStruct((S, DIM), jnp.float32),
    )(xf, Ws1, Ws3, Ws2)


def _make_meta(counts):
    off = jnp.concatenate([jnp.zeros((1,), jnp.int32),
                           jnp.cumsum(counts).astype(jnp.int32)])
    t_ids = jnp.arange(T_TILES, dtype=jnp.int32)[:, None]
    e_ids = jnp.arange(NUM_EXPERTS, dtype=jnp.int32)[None, :]
    lo = t_ids * TM
    hi = lo + TM
    st = off[:-1][None, :]
    en = off[1:][None, :]
    valid = (en > lo) & (st < hi) & (en > st)
    vflat = valid.reshape(-1)
    posq = jnp.cumsum(vflat.astype(jnp.int32)) - 1
    nvalid = jnp.sum(vflat.astype(jnp.int32))
    dump = jnp.where(vflat, posq, NUM_W)
    tq = jnp.broadcast_to(t_ids, valid.shape).reshape(-1)
    eq = jnp.broadcast_to(e_ids, valid.shape).reshape(-1)
    rsq = (jnp.maximum(st, lo) - lo).reshape(-1)
    req = (jnp.minimum(en, hi) - lo).reshape(-1)

    z = jnp.zeros((NUM_W,), jnp.int32)
    tiles = z.at[dump].set(tq, mode="drop")
    exps = z.at[dump].set(eq, mode="drop")
    rss = z.at[dump].set(rsq, mode="drop")
    res = z.at[dump].set(req, mode="drop")

    wids = jnp.arange(NUM_W, dtype=jnp.int32)
    pad = wids >= nvalid
    e_last = jnp.max(jnp.where(vflat, eq, 0))
    tiles = jnp.where(pad, T_TILES - 1, tiles)
    exps = jnp.where(pad, e_last, exps)
    rss = jnp.where(pad, 0, rss)
    res = jnp.where(pad, 0, res)
    first = (wids == 0) | (tiles != jnp.roll(tiles, 1))
    first = jnp.where(pad, False, first).astype(jnp.int32)
    return jnp.stack([tiles, exps, rss, res, first])


def kernel(x, Wg, W1, W3, W2, Ws1, Ws3, Ws2):
    xf = x.reshape(-1, DIM)

    wpair, epair, aux = _router(xf, Wg)

    e_flat = epair.reshape(-1)
    order = jnp.argsort(e_flat)
    tok = order // TOP_K
    counts = jnp.sum(e_flat[None, :] == jnp.arange(NUM_EXPERTS,
                                                   dtype=jnp.int32)[:, None],
                     axis=1).astype(jnp.int32)
    meta = _make_meta(counts)

    xs = xf[tok]
    wrow = wpair.reshape(-1)[order]

    ys = _gmm(meta, xs, wrow, W1, W3, W2)
    shared = _shared(xf, Ws1, Ws3, Ws2)

    pos = jnp.zeros((G,), jnp.int32).at[order].set(
        jnp.arange(G, dtype=jnp.int32))
    pos = pos.reshape(S, TOP_K)
    final = ys[pos[:, 0]] + ys[pos[:, 1]] + shared
    return final.reshape(x.shape), aux.reshape(())


# speedup vs baseline: 2.0508x; 1.0084x over previous
import functools

import jax
import jax.numpy as jnp
from jax.experimental import pallas as pl
from jax.experimental.pallas import tpu as pltpu

DIM = 768
NUM_EXPERTS = 8
TOP_K = 2
HID = 2058
S = 2048
G = S * TOP_K
TM = 256
T_TILES = G // TM
NUM_W = T_TILES + NUM_EXPERTS - 1


def _router_body(x_ref, wg_ref, wpair_ref, epair_ref, aux_ref):
    x = x_ref[...]
    logits = jnp.dot(x, wg_ref[...], preferred_element_type=jnp.float32)
    m = jnp.max(logits, axis=1, keepdims=True)
    e = jnp.exp(logits - m)
    p = e / jnp.sum(e, axis=1, keepdims=True)

    idx8 = jax.lax.broadcasted_iota(jnp.int32, p.shape, 1)
    m0 = jnp.max(p, axis=1, keepdims=True)
    i0 = jnp.min(jnp.where(p == m0, idx8, NUM_EXPERTS), axis=1, keepdims=True)
    pm = jnp.where(idx8 == i0, -1.0, p)
    m1 = jnp.max(pm, axis=1, keepdims=True)
    i1 = jnp.min(jnp.where(pm == m1, idx8, NUM_EXPERTS), axis=1, keepdims=True)

    s = m0 + m1
    wpair_ref[:, 0:1] = m0 / s
    wpair_ref[:, 1:2] = m1 / s
    epair_ref[:, 0:1] = i0
    epair_ref[:, 1:2] = i1

    importance = jnp.mean(p, axis=0, keepdims=True)
    load = jnp.mean((idx8 == i0).astype(jnp.float32), axis=0, keepdims=True)
    aux_ref[...] = NUM_EXPERTS * jnp.sum(
        importance * load, axis=1, keepdims=True)


def _router(xf, Wg):
    return pl.pallas_call(
        _router_body,
        out_shape=(
            jax.ShapeDtypeStruct((S, TOP_K), jnp.float32),
            jax.ShapeDtypeStruct((S, TOP_K), jnp.int32),
            jax.ShapeDtypeStruct((1, 1), jnp.float32),
        ),
    )(xf, Wg)


def _gmm_body(meta_ref, xs_ref, wr_ref, w1_ref, w3_ref, w2_ref, ys_ref):
    w = pl.program_id(0)
    rs = meta_ref[2, w]
    re_ = meta_ref[3, w]
    fst = meta_ref[4, w]

    @pl.when(fst == 1)
    def _():
        ys_ref[...] = jnp.zeros_like(ys_ref)

    @pl.when(re_ > rs)
    def _():
        x = xs_ref[...].astype(jnp.bfloat16)
        a = jnp.dot(x, w1_ref[0].astype(jnp.bfloat16),
                    preferred_element_type=jnp.float32)
        b = jnp.dot(x, w3_ref[0].astype(jnp.bfloat16),
                    preferred_element_type=jnp.float32)
        h = a * (1.0 / (1.0 + jnp.exp(-a))) * b
        y = jnp.dot(h.astype(jnp.bfloat16), w2_ref[0].astype(jnp.bfloat16),
                    preferred_element_type=jnp.float32)
        y = y * wr_ref[...]
        rows = jax.lax.broadcasted_iota(jnp.int32, (TM, 1), 0)
        mask = (rows >= rs) & (rows < re_)
        ys_ref[...] += jnp.where(mask, y, 0.0)


def _gmm(meta, xs, wrow, W1, W3, W2):
    grid_spec = pltpu.PrefetchScalarGridSpec(
        num_scalar_prefetch=1,
        grid=(NUM_W,),
        in_specs=[
            pl.BlockSpec((TM, DIM), lambda w, m: (m[0, w], 0)),
            pl.BlockSpec((TM, 1), lambda w, m: (m[0, w], 0)),
            pl.BlockSpec((1, DIM, HID), lambda w, m: (m[1, w], 0, 0)),
            pl.BlockSpec((1, DIM, HID), lambda w, m: (m[1, w], 0, 0)),
            pl.BlockSpec((1, HID, DIM), lambda w, m: (m[1, w], 0, 0)),
        ],
        out_specs=pl.BlockSpec((TM, DIM), lambda w, m: (m[0, w], 0)),
    )
    return pl.pallas_call(
        _gmm_body,
        grid_spec=grid_spec,
        out_shape=jax.ShapeDtypeStruct((G, DIM), jnp.float32),
        compiler_params=pltpu.CompilerParams(
            dimension_semantics=("arbitrary",)),
    )(meta, xs, wrow.reshape(G, 1), W1, W3, W2)


def _shared_body(x_ref, w1_ref, w3_ref, w2_ref, o_ref):
    x = x_ref[...].astype(jnp.bfloat16)
    a = jnp.dot(x, w1_ref[...].astype(jnp.bfloat16),
                preferred_element_type=jnp.float32)
    b = jnp.dot(x, w3_ref[...].astype(jnp.bfloat16),
                preferred_element_type=jnp.float32)
    h = a * (1.0 / (1.0 + jnp.exp(-a))) * b
    o_ref[...] = jnp.dot(h.astype(jnp.bfloat16), w2_ref[...].astype(jnp.bfloat16),
                         preferred_element_type=jnp.float32)


def _shared(xf, Ws1, Ws3, Ws2):
    nt = S // TM
    return pl.pallas_call(
        _shared_body,
        grid=(nt,),
        in_specs=[
            pl.BlockSpec((TM, DIM), lambda t: (t, 0)),
            pl.BlockSpec((DIM, HID), lambda t: (0, 0)),
            pl.BlockSpec((DIM, HID), lambda t: (0, 0)),
            pl.BlockSpec((HID, DIM), lambda t: (0, 0)),
        ],
        out_specs=pl.BlockSpec((TM, DIM), lambda t: (t, 0)),
        out_shape=jax.ShapeDtypeStruct((S, DIM), jnp.float32),
    )(xf, Ws1, Ws3, Ws2)


def _make_meta(counts):
    off = jnp.concatenate([jnp.zeros((1,), jnp.int32),
                           jnp.cumsum(counts).astype(jnp.int32)])
    t_ids = jnp.arange(T_TILES, dtype=jnp.int32)[:, None]
    e_ids = jnp.arange(NUM_EXPERTS, dtype=jnp.int32)[None, :]
    lo = t_ids * TM
    hi = lo + TM
    st = off[:-1][None, :]
    en = off[1:][None, :]
    valid = (en > lo) & (st < hi) & (en > st)
    vflat = valid.reshape(-1)
    posq = jnp.cumsum(vflat.astype(jnp.int32)) - 1
    nvalid = jnp.sum(vflat.astype(jnp.int32))
    dump = jnp.where(vflat, posq, NUM_W)
    tq = jnp.broadcast_to(t_ids, valid.shape).reshape(-1)
    eq = jnp.broadcast_to(e_ids, valid.shape).reshape(-1)
    rsq = (jnp.maximum(st, lo) - lo).reshape(-1)
    req = (jnp.minimum(en, hi) - lo).reshape(-1)

    z = jnp.zeros((NUM_W,), jnp.int32)
    tiles = z.at[dump].set(tq, mode="drop")
    exps = z.at[dump].set(eq, mode="drop")
    rss = z.at[dump].set(rsq, mode="drop")
    res = z.at[dump].set(req, mode="drop")

    wids = jnp.arange(NUM_W, dtype=jnp.int32)
    pad = wids >= nvalid
    e_last = jnp.max(jnp.where(vflat, eq, 0))
    tiles = jnp.where(pad, T_TILES - 1, tiles)
    exps = jnp.where(pad, e_last, exps)
    rss = jnp.where(pad, 0, rss)
    res = jnp.where(pad, 0, res)
    first = (wids == 0) | (tiles != jnp.roll(tiles, 1))
    first = jnp.where(pad, False, first).astype(jnp.int32)
    return jnp.stack([tiles, exps, rss, res, first])


def kernel(x, Wg, W1, W3, W2, Ws1, Ws3, Ws2):
    xf = x.reshape(-1, DIM)

    wpair, epair, aux = _router(xf, Wg)

    e_flat = epair.reshape(-1)
    order = jnp.argsort(e_flat)
    tok = order // TOP_K
    counts = jnp.sum(e_flat[None, :] == jnp.arange(NUM_EXPERTS,
                                                   dtype=jnp.int32)[:, None],
                     axis=1).astype(jnp.int32)
    meta = _make_meta(counts)

    xs = xf[tok]
    wrow = wpair.reshape(-1)[order]

    ys = _gmm(meta, xs, wrow, W1, W3, W2)
    shared = _shared(xf, Ws1, Ws3, Ws2)

    pos = jnp.zeros((G,), jnp.int32).at[order].set(
        jnp.arange(G, dtype=jnp.int32))
    pos = pos.reshape(S, TOP_K)
    final = ys[pos[:, 0]] + ys[pos[:, 1]] + shared
    return final.reshape(x.shape), aux.reshape(())


# speedup vs baseline: 2.1258x; 1.0365x over previous
import functools

import jax
import jax.numpy as jnp
from jax import lax
from jax.experimental import pallas as pl
from jax.experimental.pallas import tpu as pltpu

DIM = 768
NUM_EXPERTS = 8
TOP_K = 2
HID = 2058
S = 2048
G = S * TOP_K
TM = 256
T_TILES = G // TM
NUM_W = T_TILES + NUM_EXPERTS - 1


def _router_body(x_ref, wg_ref, aux_ref, pos0_ref, pos1_ref,
                 w0p_ref, w1p_ref, meta_ref):
    x = x_ref[...]
    logits = jnp.dot(x, wg_ref[...], preferred_element_type=jnp.float32)
    m = jnp.max(logits, axis=1, keepdims=True)
    e = jnp.exp(logits - m)
    p = e / jnp.sum(e, axis=1, keepdims=True)

    idx8 = lax.broadcasted_iota(jnp.int32, p.shape, 1)
    m0 = jnp.max(p, axis=1, keepdims=True)
    i0 = jnp.min(jnp.where(p == m0, idx8, NUM_EXPERTS), axis=1, keepdims=True)
    pm = jnp.where(idx8 == i0, -1.0, p)
    m1 = jnp.max(pm, axis=1, keepdims=True)
    i1 = jnp.min(jnp.where(pm == m1, idx8, NUM_EXPERTS), axis=1, keepdims=True)

    ssum = m0 + m1
    w0p_ref[...] = jnp.broadcast_to(m0 / ssum, (S, 16))
    w1p_ref[...] = jnp.broadcast_to(m1 / ssum, (S, 16))

    importance = jnp.mean(p, axis=0, keepdims=True)
    load = jnp.mean((idx8 == i0).astype(jnp.float32), axis=0, keepdims=True)
    aux_ref[...] = NUM_EXPERTS * jnp.sum(
        importance * load, axis=1, keepdims=True)

    oh0 = (idx8 == i0).astype(jnp.float32)
    oh1 = (idx8 == i1).astype(jnp.float32)
    oh = oh0 + oh1
    c = oh
    d = 1
    while d < S:
        c = c + jnp.concatenate(
            [jnp.zeros((d, NUM_EXPERTS), jnp.float32), c[:-d]], axis=0)
        d *= 2
    c_ex = c - oh
    counts = jnp.sum(oh, axis=0, keepdims=True)
    oc = counts
    d = 1
    while d < NUM_EXPERTS:
        oc = oc + jnp.concatenate(
            [jnp.zeros((1, d), jnp.float32), oc[:, :-d]], axis=1)
        d *= 2
    off = oc - counts
    base = c_ex + off
    pos0 = jnp.sum(oh0 * base, axis=1, keepdims=True)
    pos1 = jnp.sum(oh1 * base, axis=1, keepdims=True)
    pos0_ref[...] = pos0.astype(jnp.int32)
    pos1_ref[...] = pos1.astype(jnp.int32)

    st = jnp.broadcast_to(off, (T_TILES, NUM_EXPERTS))
    en = jnp.broadcast_to(off + counts, (T_TILES, NUM_EXPERTS))
    t_col = lax.broadcasted_iota(
        jnp.int32, (T_TILES, NUM_EXPERTS), 0).astype(jnp.float32)
    e_col = lax.broadcasted_iota(
        jnp.int32, (T_TILES, NUM_EXPERTS), 1).astype(jnp.float32)
    lo = t_col * TM
    hi = lo + TM
    valid = (en > lo) & (st < hi) & (en > st)
    vf = valid.astype(jnp.float32)
    rs_loc = jnp.maximum(st, lo) - lo
    re_loc = jnp.minimum(en, hi) - lo
    ri = vf
    d = 1
    while d < NUM_EXPERTS:
        ri = ri + jnp.concatenate(
            [jnp.zeros((T_TILES, d), jnp.float32), ri[:, :-d]], axis=1)
        d *= 2
    row_tot = jnp.sum(vf, axis=1, keepdims=True)
    rt = row_tot
    d = 1
    while d < T_TILES:
        rt = rt + jnp.concatenate(
            [jnp.zeros((d, 1), jnp.float32), rt[:-d]], axis=0)
        d *= 2
    posq = (ri - vf) + (rt - row_tot)
    nvalid = jnp.sum(vf)
    e_last = jnp.max(jnp.where(valid, e_col, -1.0))

    prev_tile = jnp.float32(-1.0)
    for w in range(NUM_W):
        selm = jnp.where(valid & (posq == w), 1.0, 0.0)
        pad = jnp.float32(w) >= nvalid
        tile_w = jnp.where(pad, T_TILES - 1.0, jnp.sum(selm * t_col))
        exp_w = jnp.where(pad, e_last, jnp.sum(selm * e_col))
        rs_w = jnp.where(pad, 0.0, jnp.sum(selm * rs_loc))
        re_w = jnp.where(pad, 0.0, jnp.sum(selm * re_loc))
        first_w = jnp.where(tile_w != prev_tile, 1, 0)
        meta_ref[0, w] = tile_w.astype(jnp.int32)
        meta_ref[1, w] = exp_w.astype(jnp.int32)
        meta_ref[2, w] = rs_w.astype(jnp.int32)
        meta_ref[3, w] = re_w.astype(jnp.int32)
        meta_ref[4, w] = first_w
        prev_tile = tile_w


def _router(xf, Wg):
    return pl.pallas_call(
        _router_body,
        out_shape=(
            jax.ShapeDtypeStruct((1, 1), jnp.float32),
            jax.ShapeDtypeStruct((S, 1), jnp.int32),
            jax.ShapeDtypeStruct((S, 1), jnp.int32),
            jax.ShapeDtypeStruct((S, 16), jnp.float32),
            jax.ShapeDtypeStruct((S, 16), jnp.float32),
            jax.ShapeDtypeStruct((5, NUM_W), jnp.int32),
        ),
        out_specs=(
            pl.BlockSpec(memory_space=pltpu.VMEM),
            pl.BlockSpec(memory_space=pltpu.VMEM),
            pl.BlockSpec(memory_space=pltpu.VMEM),
            pl.BlockSpec(memory_space=pltpu.VMEM),
            pl.BlockSpec(memory_space=pltpu.VMEM),
            pl.BlockSpec(memory_space=pltpu.SMEM),
        ),
    )(xf, Wg)


def _gmm_body(meta_ref, xs_ref, wr_ref, w1_ref, w3_ref, w2_ref, ys_ref):
    w = pl.program_id(0)
    rs = meta_ref[2, w]
    re_ = meta_ref[3, w]
    fst = meta_ref[4, w]

    @pl.when(fst == 1)
    def _():
        ys_ref[...] = jnp.zeros_like(ys_ref)

    @pl.when(re_ > rs)
    def _():
        x = xs_ref[...].astype(jnp.bfloat16)
        a = jnp.dot(x, w1_ref[0].astype(jnp.bfloat16),
                    preferred_element_type=jnp.float32)
        b = jnp.dot(x, w3_ref[0].astype(jnp.bfloat16),
                    preferred_element_type=jnp.float32)
        h = a * (1.0 / (1.0 + jnp.exp(-a))) * b
        y = jnp.dot(h.astype(jnp.bfloat16), w2_ref[0].astype(jnp.bfloat16),
                    preferred_element_type=jnp.float32)
        y = y * wr_ref[:, 0:1]
        rows = lax.broadcasted_iota(jnp.int32, (TM, 1), 0)
        mask = (rows >= rs) & (rows < re_)
        ys_ref[...] += jnp.where(mask, y, 0.0)


def _gmm(meta, xs, wpad, W1, W3, W2):
    grid_spec = pltpu.PrefetchScalarGridSpec(
        num_scalar_prefetch=1,
        grid=(NUM_W,),
        in_specs=[
            pl.BlockSpec((TM, DIM), lambda w, m: (m[0, w], 0)),
            pl.BlockSpec((TM, 16), lambda w, m: (m[0, w], 0)),
            pl.BlockSpec((1, DIM, HID), lambda w, m: (m[1, w], 0, 0)),
            pl.BlockSpec((1, DIM, HID), lambda w, m: (m[1, w], 0, 0)),
            pl.BlockSpec((1, HID, DIM), lambda w, m: (m[1, w], 0, 0)),
        ],
        out_specs=pl.BlockSpec((TM, DIM), lambda w, m: (m[0, w], 0)),
    )
    return pl.pallas_call(
        _gmm_body,
        grid_spec=grid_spec,
        out_shape=jax.ShapeDtypeStruct((G, DIM), jnp.float32),
        compiler_params=pltpu.CompilerParams(
            dimension_semantics=("arbitrary",)),
    )(meta, xs, wpad, W1, W3, W2)


def _shared_body(x_ref, w1_ref, w3_ref, w2_ref, o_ref):
    x = x_ref[...].astype(jnp.bfloat16)
    a = jnp.dot(x, w1_ref[...].astype(jnp.bfloat16),
                preferred_element_type=jnp.float32)
    b = jnp.dot(x, w3_ref[...].astype(jnp.bfloat16),
                preferred_element_type=jnp.float32)
    h = a * (1.0 / (1.0 + jnp.exp(-a))) * b
    o_ref[...] = jnp.dot(h.astype(jnp.bfloat16), w2_ref[...].astype(jnp.bfloat16),
                         preferred_element_type=jnp.float32)


def _shared(xf, Ws1, Ws3, Ws2):
    nt = S // TM
    return pl.pallas_call(
        _shared_body,
        grid=(nt,),
        in_specs=[
            pl.BlockSpec((TM, DIM), lambda t: (t, 0)),
            pl.BlockSpec((DIM, HID), lambda t: (0, 0)),
            pl.BlockSpec((DIM, HID), lambda t: (0, 0)),
            pl.BlockSpec((HID, DIM), lambda t: (0, 0)),
        ],
        out_specs=pl.BlockSpec((TM, DIM), lambda t: (t, 0)),
        out_shape=jax.ShapeDtypeStruct((S, DIM), jnp.float32),
    )(xf, Ws1, Ws3, Ws2)


def kernel(x, Wg, W1, W3, W2, Ws1, Ws3, Ws2):
    xf = x.reshape(-1, DIM)

    aux, pos0, pos1, w0p, w1p, meta = _router(xf, Wg)
    pos0f = pos0.reshape(S)
    pos1f = pos1.reshape(S)

    xs = jnp.zeros((G, DIM), jnp.float32).at[pos0f].set(xf).at[pos1f].set(xf)
    wpad = (jnp.zeros((G, 16), jnp.float32)
            .at[pos0f].set(w0p).at[pos1f].set(w1p))

    ys = _gmm(meta, xs, wpad, W1, W3, W2)
    shared = _shared(xf, Ws1, Ws3, Ws2)

    final = ys[pos0f] + ys[pos1f] + shared
    return final.reshape(x.shape), aux.reshape(())
